# baseline (speedup 1.0000x reference)
import jax
import jax.numpy as jnp
from jax import lax
from jax.experimental import pallas as pl
from jax.experimental.pallas import tpu as pltpu
from jax.experimental.pallas import tpu_sc as plsc

B, S, D = 4, 4096, 2048
N = B * S
NW = 32
ROWS_PER_W = N // NW
SL_PER_W = ROWS_PER_W // B
CHUNK_SL = 2
NCHUNK = SL_PER_W // CHUNK_SL
NBUF = 4
NGRP = NCHUNK // NBUF


def _gather_body(idx_hbm, table_hbm, out_hbm, idx_v, bufs, gsems, osems):
    wid = lax.axis_index("s") * 2 + lax.axis_index("c")
    base = wid * ROWS_PER_W
    sl_base = wid * SL_PER_W
    pltpu.sync_copy(idx_hbm.at[pl.ds(base, ROWS_PER_W)], idx_v)

    def gather(c, b):
        pltpu.async_copy(
            table_hbm.at[idx_v.at[pl.ds(c * CHUNK_SL * B, CHUNK_SL * B)]],
            bufs[b], gsems[b])

    def wait_gather(b):
        pltpu.make_async_copy(table_hbm.at[idx_v.at[pl.ds(0, CHUNK_SL * B)]],
                              bufs[b], gsems[b]).wait()

    def writeout(c, b):
        for j in range(CHUNK_SL):
            pltpu.async_copy(bufs[b].at[pl.ds(j * B, B)],
                             out_hbm.at[sl_base + c * CHUNK_SL + j],
                             osems[b])

    def wait_writeout(b):
        for j in range(CHUNK_SL):
            pltpu.make_async_copy(bufs[b].at[pl.ds(j * B, B)],
                                  out_hbm.at[sl_base], osems[b]).wait()

    for b in range(NBUF):
        gather(b, b)

    def step(g, _):
        for b in range(NBUF):
            c = g * NBUF + b
            wait_gather(b)
            writeout(c, b)

            @pl.when(c + NBUF < NCHUNK)
            def _():
                wait_writeout(b)
                gather(c + NBUF, b)
        return 0

    lax.fori_loop(0, NGRP, step, 0)
    for b in range(NBUF):
        wait_writeout(b)


@jax.jit
def _embed_gather(idx, embed_table):
    mesh = plsc.VectorSubcoreMesh(core_axis_name="c", subcore_axis_name="s")
    return pl.kernel(
        _gather_body,
        out_type=jax.ShapeDtypeStruct((S, B, D), jnp.float32),
        mesh=mesh,
        scratch_types=[
            pltpu.VMEM((ROWS_PER_W,), jnp.int32),
            [pltpu.VMEM((CHUNK_SL * B, D), jnp.float32) for _ in range(NBUF)],
            [pltpu.SemaphoreType.DMA for _ in range(NBUF)],
            [pltpu.SemaphoreType.DMA for _ in range(NBUF)],
        ],
    )(idx, embed_table)


def kernel(tokens, embed_table):
    idx = tokens.astype(jnp.int32).T.reshape(-1)
    return _embed_gather(idx, embed_table)


# speedup vs baseline: 1.0142x; 1.0142x over previous
import jax
import jax.numpy as jnp
from jax import lax
from jax.experimental import pallas as pl
from jax.experimental.pallas import tpu as pltpu
from jax.experimental.pallas import tpu_sc as plsc

B, S, D = 4, 4096, 2048
NW = 32
W_PER_B = NW // B
S_PER_W = S // W_PER_B
CHUNK = 8
NCHUNK = S_PER_W // CHUNK
NBUF = 4
NGRP = NCHUNK // NBUF


def _gather_body(tokens_hbm, table_hbm, out_hbm, idx_v, bufs, gsems, osems):
    wid = lax.axis_index("s") * 2 + lax.axis_index("c")
    b_row = lax.rem(wid, B)
    s0 = lax.div(wid, B) * S_PER_W
    pltpu.sync_copy(tokens_hbm.at[b_row, pl.ds(s0, S_PER_W)], idx_v)

    def gather(c, b):
        pltpu.async_copy(
            table_hbm.at[idx_v.at[pl.ds(c * CHUNK, CHUNK)]],
            bufs[b], gsems[b])

    def wait_gather(b):
        pltpu.make_async_copy(table_hbm.at[idx_v.at[pl.ds(0, CHUNK)]],
                              bufs[b], gsems[b]).wait()

    def writeout(c, b):
        pltpu.async_copy(bufs[b], out_hbm.at[pl.ds(s0 + c * CHUNK, CHUNK),
                                             b_row], osems[b])

    def wait_writeout(b):
        pltpu.make_async_copy(bufs[b], out_hbm.at[pl.ds(s0, CHUNK), b_row],
                              osems[b]).wait()

    for b in range(NBUF):
        gather(b, b)

    def step(g, _):
        for b in range(NBUF):
            c = g * NBUF + b
            wait_gather(b)
            writeout(c, b)

            @pl.when(c + NBUF < NCHUNK)
            def _():
                wait_writeout(b)
                gather(c + NBUF, b)
        return 0

    lax.fori_loop(0, NGRP, step, 0)
    for b in range(NBUF):
        wait_writeout(b)


@jax.jit
def _embed_gather(tokens, embed_table):
    mesh = plsc.VectorSubcoreMesh(core_axis_name="c", subcore_axis_name="s")
    return pl.kernel(
        _gather_body,
        out_type=jax.ShapeDtypeStruct((S, B, D), jnp.float32),
        mesh=mesh,
        scratch_types=[
            pltpu.VMEM((S_PER_W,), jnp.int32),
            [pltpu.VMEM((CHUNK, D), jnp.float32) for _ in range(NBUF)],
            [pltpu.SemaphoreType.DMA for _ in range(NBUF)],
            [pltpu.SemaphoreType.DMA for _ in range(NBUF)],
        ],
    )(tokens, embed_table)


def kernel(tokens, embed_table):
    return _embed_gather(tokens.astype(jnp.int32), embed_table)
